# baseline (speedup 1.0000x reference)
import jax
import jax.numpy as jnp
from jax import lax
from jax.experimental import pallas as pl
from jax.experimental.pallas import tpu as pltpu
from jax.experimental.pallas import tpu_sc as plsc

BS = 1024
D = 64
N_CAND = 501
NT = 512
NW = 32
B_PER = BS // NW
CHUNK = 256
NSTREAM = 128
NQ = NT // CHUNK
NSTEP = B_PER * NQ
NLANE = 16
NC = D // NLANE


def _sc_body(tabE, tabR, sub_h, rel_h, ent_h, dRI_h, out_h,
             sub_v, rel_v, es, rr, dRI, aRI,
             idx_v, rows, scores, sem0, semg0, semg1):
    wid = lax.axis_index("s") * 2 + lax.axis_index("c")
    base_b = wid * B_PER

    pltpu.sync_copy(sub_h.at[pl.ds(base_b, B_PER)], sub_v)
    pltpu.sync_copy(rel_h.at[pl.ds(base_b, B_PER)], rel_v)
    pltpu.sync_copy(dRI_h.at[pl.ds(base_b, B_PER)], dRI)
    pltpu.async_copy(tabE.at[sub_v], es, sem0).wait()
    pltpu.async_copy(tabR.at[rel_v], rr, sem0).wait()

    def prep(i, carry):
        for c in range(NC):
            s = pl.ds(c * NLANE, NLANE)
            si = pl.ds(D + c * NLANE, NLANE)
            er = es[i, s]
            ei = es[i, si]
            dr = dRI[i, s]
            di = dRI[i, si]
            aRI[i, s] = er * dr - ei * di + rr[i, s]
            aRI[i, si] = er * di + ei * dr + rr[i, si]
        return carry

    lax.fori_loop(0, B_PER, prep, 0)

    lane = lax.iota(jnp.int32, NLANE)
    bufs = ((rows.at[0], idx_v.at[0], semg0), (rows.at[1], idx_v.at[1], semg1))

    def issue(step, db):
        b_local = lax.shift_right_logical(step, 1)
        q = lax.bitwise_and(step, NQ - 1)
        rbuf, ibuf, sem = bufs[db]
        for k in range(CHUNK // NSTREAM):
            pltpu.sync_copy(
                ent_h.at[(base_b + b_local) * 4 + q * 2 + k], ibuf.at[k])
            pltpu.async_copy(tabE.at[ibuf.at[k]],
                             rbuf.at[pl.ds(k * NSTREAM, NSTREAM)], sem)

    issue(0, 0)
    issue(1, 1)

    def compute(step, db):
        b_local = lax.shift_right_logical(step, 1)
        q = lax.bitwise_and(step, NQ - 1)
        rbuf, ibuf, sem = bufs[db]
        for k in range(CHUNK // NSTREAM):
            pltpu.make_async_copy(
                tabE.at[ibuf.at[k]],
                rbuf.at[pl.ds(k * NSTREAM, NSTREAM)], sem).wait()

        ars = [aRI[b_local, pl.ds(c * NLANE, NLANE)] for c in range(NC)]
        ais = [aRI[b_local, pl.ds(D + c * NLANE, NLANE)] for c in range(NC)]
        drs = [dRI[b_local, pl.ds(c * NLANE, NLANE)] for c in range(NC)]
        dis = [dRI[b_local, pl.ds(D + c * NLANE, NLANE)] for c in range(NC)]

        def group(h, gcarry):
            vec = jnp.zeros((NLANE,), jnp.float32)
            for jj in range(NLANE):
                j = h * NLANE + jj
                part = []
                for c in range(NC):
                    er = rbuf[j, pl.ds(c * NLANE, NLANE)]
                    ei = rbuf[j, pl.ds(D + c * NLANE, NLANE)]
                    t1 = (ars[c] + ei * dis[c]) - er * drs[c]
                    t2 = (ais[c] + er * dis[c]) + ei * drs[c]
                    part.append(jnp.abs(t1) + jnp.abs(t2))
                acc = (part[0] + part[1]) + (part[2] + part[3])
                vec = jnp.where(lane == jj,
                                jnp.full((NLANE,), jnp.sum(acc)), vec)
            row = q * 2 + lax.shift_right_logical(h, 3)
            scores[row, pl.ds(lax.bitwise_and(h, 7) * NLANE, NLANE)] = vec
            return gcarry

        lax.fori_loop(0, CHUNK // NLANE, group, 0)

        @pl.when(q == NQ - 1)
        def _():
            pltpu.sync_copy(scores,
                            out_h.at[pl.ds((base_b + b_local) * 4, 4), :])

    def step_pair(t, carry):
        for db in range(2):
            s = 2 * t + db
            compute(s, db)

            @pl.when(s + 2 < NSTEP)
            def _():
                issue(s + 2, db)

        return carry

    lax.fori_loop(0, NSTEP // 2, step_pair, 0)


def _sc_scores(tabE, tabR, sub, rel, ent4, dRI):
    mesh = plsc.VectorSubcoreMesh(core_axis_name="c", subcore_axis_name="s")
    f = pl.kernel(
        _sc_body,
        out_type=jax.ShapeDtypeStruct((BS * 4, 2 * D), jnp.float32),
        mesh=mesh,
        compiler_params=pltpu.CompilerParams(needs_layout_passes=False),
        scratch_types=[
            pltpu.VMEM((B_PER,), jnp.int32),
            pltpu.VMEM((B_PER,), jnp.int32),
            pltpu.VMEM((B_PER, 2 * D), jnp.float32),
            pltpu.VMEM((B_PER, 2 * D), jnp.float32),
            pltpu.VMEM((B_PER, 2 * D), jnp.float32),
            pltpu.VMEM((B_PER, 2 * D), jnp.float32),
            pltpu.VMEM((2, 2, NSTREAM), jnp.int32),
            pltpu.VMEM((2, CHUNK, 2 * D), jnp.float32),
            pltpu.VMEM((4, 2 * D), jnp.float32),
            pltpu.SemaphoreType.DMA,
            pltpu.SemaphoreType.DMA,
            pltpu.SemaphoreType.DMA,
        ],
    )
    return f(tabE, tabR, sub, rel, ent4, dRI)


def _pack_body(rt_ref, it_ref, o_ref):
    i0 = lax.broadcasted_iota(jnp.int32, (D, D), 0)
    i1 = lax.broadcasted_iota(jnp.int32, (D, D), 1)
    eye = (i0 == i1).astype(jnp.float32)
    dn = (((0,), (0,)), ((), ()))
    o_ref[:, :D] = lax.dot_general(rt_ref[...], eye, dn,
                                   preferred_element_type=jnp.float32)
    o_ref[:, D:] = lax.dot_general(it_ref[...], eye, dn,
                                   preferred_element_type=jnp.float32)


def _pack_tables(real, img, blk):
    n = real.shape[0]
    grid = (n + blk - 1) // blk
    return pl.pallas_call(
        _pack_body,
        grid=(grid,),
        in_specs=[
            pl.BlockSpec((D, blk), lambda i: (0, i)),
            pl.BlockSpec((D, blk), lambda i: (0, i)),
        ],
        out_specs=pl.BlockSpec((blk, 2 * D), lambda i: (i, 0)),
        out_shape=jax.ShapeDtypeStruct((n, 2 * D), jnp.float32),
    )(real.T, img.T)


def _trig_body(day_ref, w1_ref, w2_ref, dri_ref):
    day = day_ref[...]
    dri_ref[:, :D] = jnp.cos(w2_ref[...] * day)
    dri_ref[:, D:] = jnp.sin(w1_ref[...] * day)


def _loss_body(x_ref, out_ref):
    x = x_ref[...]
    col = lax.broadcasted_iota(jnp.int32, (BS, NT), 1)
    valid = col < N_CAND
    xm = jnp.where(valid, x, -jnp.inf)
    m = jnp.max(xm, axis=1, keepdims=True)
    ssum = jnp.sum(jnp.where(valid, jnp.exp(xm - m), 0.0), axis=1,
                   keepdims=True)
    lse = m + jnp.log(ssum)
    pos = jnp.sum(jnp.where(col == 0, x, 0.0), axis=1, keepdims=True)
    val = jnp.sum(lse - pos) * (1.0 / BS)
    out_ref[...] = jnp.full((1, 1), val, jnp.float32)


def kernel(sub, rel, obj, year, month, day, neg, emb_E_real, emb_E_img,
           emb_R_real, emb_R_img, w1, w2):
    ent = jnp.concatenate([obj[:, None], neg], axis=1).astype(jnp.int32)
    ent = jnp.pad(ent, ((0, 0), (0, NT - N_CAND)))
    ent4 = ent.reshape(BS * 4, 2 * D)

    tabE = _pack_tables(emb_E_real, emb_E_img, 2048)
    tabR = _pack_tables(emb_R_real, emb_R_img, 1000)

    dRI = pl.pallas_call(
        _trig_body,
        out_shape=jax.ShapeDtypeStruct((BS, 2 * D), jnp.float32),
    )(day.reshape(BS, 1), w1.reshape(1, D), w2.reshape(1, D))

    scores4 = _sc_scores(tabE, tabR, sub.astype(jnp.int32),
                         rel.astype(jnp.int32), ent4, dRI)

    loss = pl.pallas_call(
        _loss_body,
        out_shape=jax.ShapeDtypeStruct((1, 1), jnp.float32),
    )(scores4.reshape(BS, NT))
    return loss[0, 0]


# speedup vs baseline: 1.0010x; 1.0010x over previous
import jax
import jax.numpy as jnp
from jax import lax
from jax.experimental import pallas as pl
from jax.experimental.pallas import tpu as pltpu
from jax.experimental.pallas import tpu_sc as plsc

BS = 1024
D = 64
N_CAND = 501
NT = 512
NW = 32
B_PER = BS // NW
CHUNK = 256
NSTREAM = 128
NQ = NT // CHUNK
NSTEP = B_PER * NQ
NLANE = 16
NC = D // NLANE


def _sc_body(tabE, tabR, sub_h, rel_h, ent_h, dRI_h, out_h,
             sub_v, rel_v, es, rr, dRI, aRI,
             idx_all, rows, scores, sem0, semg0, semg1):
    wid = lax.axis_index("s") * 2 + lax.axis_index("c")
    base_b = wid * B_PER

    pltpu.sync_copy(sub_h.at[pl.ds(base_b, B_PER)], sub_v)
    pltpu.sync_copy(rel_h.at[pl.ds(base_b, B_PER)], rel_v)
    pltpu.sync_copy(dRI_h.at[pl.ds(base_b, B_PER)], dRI)
    pltpu.sync_copy(ent_h.at[pl.ds(base_b * 4, B_PER * 4), :], idx_all)
    pltpu.async_copy(tabE.at[sub_v], es, sem0).wait()
    pltpu.async_copy(tabR.at[rel_v], rr, sem0).wait()

    def prep(i, carry):
        for c in range(NC):
            s = pl.ds(c * NLANE, NLANE)
            si = pl.ds(D + c * NLANE, NLANE)
            er = es[i, s]
            ei = es[i, si]
            dr = dRI[i, s]
            di = dRI[i, si]
            aRI[i, s] = er * dr - ei * di + rr[i, s]
            aRI[i, si] = er * di + ei * dr + rr[i, si]
        return carry

    lax.fori_loop(0, B_PER, prep, 0)

    lane = lax.iota(jnp.int32, NLANE)
    bufs = ((rows.at[0], semg0), (rows.at[1], semg1))

    def issue(step, db):
        rbuf, sem = bufs[db]
        for k in range(CHUNK // NSTREAM):
            pltpu.async_copy(tabE.at[idx_all.at[step * 2 + k]],
                             rbuf.at[pl.ds(k * NSTREAM, NSTREAM)], sem)

    issue(0, 0)
    issue(1, 1)

    def compute(step, db):
        b_local = lax.shift_right_logical(step, 1)
        q = lax.bitwise_and(step, NQ - 1)
        rbuf, sem = bufs[db]
        for k in range(CHUNK // NSTREAM):
            pltpu.make_async_copy(
                tabE.at[idx_all.at[step * 2 + k]],
                rbuf.at[pl.ds(k * NSTREAM, NSTREAM)], sem).wait()

        ars = [aRI[b_local, pl.ds(c * NLANE, NLANE)] for c in range(NC)]
        ais = [aRI[b_local, pl.ds(D + c * NLANE, NLANE)] for c in range(NC)]
        drs = [dRI[b_local, pl.ds(c * NLANE, NLANE)] for c in range(NC)]
        dis = [dRI[b_local, pl.ds(D + c * NLANE, NLANE)] for c in range(NC)]

        def group(h, gcarry):
            vec = jnp.zeros((NLANE,), jnp.float32)
            for jj in range(NLANE):
                j = h * NLANE + jj
                part = []
                for c in range(NC):
                    er = rbuf[j, pl.ds(c * NLANE, NLANE)]
                    ei = rbuf[j, pl.ds(D + c * NLANE, NLANE)]
                    t1 = (ars[c] + ei * dis[c]) - er * drs[c]
                    t2 = (ais[c] + er * dis[c]) + ei * drs[c]
                    part.append(jnp.abs(t1) + jnp.abs(t2))
                acc = (part[0] + part[1]) + (part[2] + part[3])
                vec = jnp.where(lane == jj,
                                jnp.full((NLANE,), jnp.sum(acc)), vec)
            row = b_local * 4 + q * 2 + lax.shift_right_logical(h, 3)
            scores[row, pl.ds(lax.bitwise_and(h, 7) * NLANE, NLANE)] = vec
            return gcarry

        lax.fori_loop(0, CHUNK // NLANE, group, 0)

    def step_pair(t, carry):
        for db in range(2):
            s = 2 * t + db
            compute(s, db)

            @pl.when(s + 2 < NSTEP)
            def _():
                issue(s + 2, db)

        return carry

    lax.fori_loop(0, NSTEP // 2, step_pair, 0)
    pltpu.sync_copy(scores, out_h.at[pl.ds(base_b * 4, B_PER * 4), :])


def _sc_scores(tabE, tabR, sub, rel, ent4, dRI):
    mesh = plsc.VectorSubcoreMesh(core_axis_name="c", subcore_axis_name="s")
    f = pl.kernel(
        _sc_body,
        out_type=jax.ShapeDtypeStruct((BS * 4, 2 * D), jnp.float32),
        mesh=mesh,
        compiler_params=pltpu.CompilerParams(needs_layout_passes=False),
        scratch_types=[
            pltpu.VMEM((B_PER,), jnp.int32),
            pltpu.VMEM((B_PER,), jnp.int32),
            pltpu.VMEM((B_PER, 2 * D), jnp.float32),
            pltpu.VMEM((B_PER, 2 * D), jnp.float32),
            pltpu.VMEM((B_PER, 2 * D), jnp.float32),
            pltpu.VMEM((B_PER, 2 * D), jnp.float32),
            pltpu.VMEM((B_PER * 4, NSTREAM), jnp.int32),
            pltpu.VMEM((2, CHUNK, 2 * D), jnp.float32),
            pltpu.VMEM((B_PER * 4, 2 * D), jnp.float32),
            pltpu.SemaphoreType.DMA,
            pltpu.SemaphoreType.DMA,
            pltpu.SemaphoreType.DMA,
        ],
    )
    return f(tabE, tabR, sub, rel, ent4, dRI)


def _pack_body(rt_ref, it_ref, o_ref):
    o_ref[:, :D] = rt_ref[...].T
    o_ref[:, D:] = it_ref[...].T


def _pack_tables(real, img, blk):
    n = real.shape[0]
    grid = (n + blk - 1) // blk
    return pl.pallas_call(
        _pack_body,
        grid=(grid,),
        in_specs=[
            pl.BlockSpec((D, blk), lambda i: (0, i)),
            pl.BlockSpec((D, blk), lambda i: (0, i)),
        ],
        out_specs=pl.BlockSpec((blk, 2 * D), lambda i: (i, 0)),
        out_shape=jax.ShapeDtypeStruct((n, 2 * D), jnp.float32),
    )(real.T, img.T)


def _trig_body(day_ref, w1_ref, w2_ref, dri_ref):
    day = day_ref[...]
    dri_ref[:, :D] = jnp.cos(w2_ref[...] * day)
    dri_ref[:, D:] = jnp.sin(w1_ref[...] * day)


def _loss_body(x_ref, out_ref):
    x = x_ref[...]
    col = lax.broadcasted_iota(jnp.int32, (BS, NT), 1)
    valid = col < N_CAND
    xm = jnp.where(valid, x, -jnp.inf)
    m = jnp.max(xm, axis=1, keepdims=True)
    ssum = jnp.sum(jnp.where(valid, jnp.exp(xm - m), 0.0), axis=1,
                   keepdims=True)
    lse = m + jnp.log(ssum)
    pos = jnp.sum(jnp.where(col == 0, x, 0.0), axis=1, keepdims=True)
    val = jnp.sum(lse - pos) * (1.0 / BS)
    out_ref[...] = jnp.full((1, 1), val, jnp.float32)


def kernel(sub, rel, obj, year, month, day, neg, emb_E_real, emb_E_img,
           emb_R_real, emb_R_img, w1, w2):
    ent = jnp.concatenate([obj[:, None], neg], axis=1).astype(jnp.int32)
    ent = jnp.pad(ent, ((0, 0), (0, NT - N_CAND)))
    ent4 = ent.reshape(BS * 4, 2 * D)

    tabE = _pack_tables(emb_E_real, emb_E_img, 2048)
    tabR = _pack_tables(emb_R_real, emb_R_img, 1000)

    dRI = pl.pallas_call(
        _trig_body,
        out_shape=jax.ShapeDtypeStruct((BS, 2 * D), jnp.float32),
    )(day.reshape(BS, 1), w1.reshape(1, D), w2.reshape(1, D))

    scores4 = _sc_scores(tabE, tabR, sub.astype(jnp.int32),
                         rel.astype(jnp.int32), ent4, dRI)

    loss = pl.pallas_call(
        _loss_body,
        out_shape=jax.ShapeDtypeStruct((1, 1), jnp.float32),
    )(scores4.reshape(BS, NT))
    return loss[0, 0]


# speedup vs baseline: 1.0017x; 1.0007x over previous
import jax
import jax.numpy as jnp
from jax import lax
from jax.experimental import pallas as pl
from jax.experimental.pallas import tpu as pltpu
from jax.experimental.pallas import tpu_sc as plsc

BS = 1024
D = 64
N_CAND = 501
NT = 512
NW = 32
B_PER = BS // NW
CHUNK = 128
NBUF = 4
NQ = NT // CHUNK
NSTEP = B_PER * NQ
NLANE = 16
NC = D // NLANE


def _sc_body(tabE, tabR, sub_h, rel_h, ent_h, dRI_h, out_h,
             sub_v, rel_v, es, rr, dRI, aRI,
             idx_all, rows, scores, sem0, semg0, semg1, semg2, semg3):
    wid = lax.axis_index("s") * 2 + lax.axis_index("c")
    base_b = wid * B_PER

    pltpu.sync_copy(sub_h.at[pl.ds(base_b, B_PER)], sub_v)
    pltpu.sync_copy(rel_h.at[pl.ds(base_b, B_PER)], rel_v)
    pltpu.sync_copy(dRI_h.at[pl.ds(base_b, B_PER)], dRI)
    pltpu.sync_copy(ent_h.at[pl.ds(base_b * 4, B_PER * 4), :], idx_all)
    pltpu.async_copy(tabE.at[sub_v], es, sem0).wait()
    pltpu.async_copy(tabR.at[rel_v], rr, sem0).wait()

    def prep(i, carry):
        for c in range(NC):
            s = pl.ds(c * NLANE, NLANE)
            si = pl.ds(D + c * NLANE, NLANE)
            er = es[i, s]
            ei = es[i, si]
            dr = dRI[i, s]
            di = dRI[i, si]
            aRI[i, s] = er * dr - ei * di + rr[i, s]
            aRI[i, si] = er * di + ei * dr + rr[i, si]
        return carry

    lax.fori_loop(0, B_PER, prep, 0)

    lane = lax.iota(jnp.int32, NLANE)
    bufs = ((rows.at[0], semg0), (rows.at[1], semg1),
            (rows.at[2], semg2), (rows.at[3], semg3))

    def issue(step, db):
        rbuf, sem = bufs[db]
        pltpu.async_copy(tabE.at[idx_all.at[step]], rbuf, sem)

    for _p in range(NBUF):
        issue(_p, _p)

    def compute(step, db):
        b_local = lax.shift_right_logical(step, 2)
        q = lax.bitwise_and(step, NQ - 1)
        rbuf, sem = bufs[db]
        pltpu.make_async_copy(tabE.at[idx_all.at[step]], rbuf, sem).wait()

        ars = [aRI[b_local, pl.ds(c * NLANE, NLANE)] for c in range(NC)]
        ais = [aRI[b_local, pl.ds(D + c * NLANE, NLANE)] for c in range(NC)]
        drs = [dRI[b_local, pl.ds(c * NLANE, NLANE)] for c in range(NC)]
        dis = [dRI[b_local, pl.ds(D + c * NLANE, NLANE)] for c in range(NC)]

        def group(h, gcarry):
            vec = jnp.zeros((NLANE,), jnp.float32)
            for jj in range(NLANE):
                j = h * NLANE + jj
                part = []
                for c in range(NC):
                    er = rbuf[j, pl.ds(c * NLANE, NLANE)]
                    ei = rbuf[j, pl.ds(D + c * NLANE, NLANE)]
                    t1 = (ars[c] + ei * dis[c]) - er * drs[c]
                    t2 = (ais[c] + er * dis[c]) + ei * drs[c]
                    part.append(jnp.abs(t1) + jnp.abs(t2))
                acc = (part[0] + part[1]) + (part[2] + part[3])
                vec = jnp.where(lane == jj,
                                jnp.full((NLANE,), jnp.sum(acc)), vec)
            row = b_local * 4 + q
            scores[row, pl.ds(h * NLANE, NLANE)] = vec
            return gcarry

        lax.fori_loop(0, CHUNK // NLANE, group, 0)

    def step_quad(t, carry):
        for db in range(NBUF):
            s = NBUF * t + db
            compute(s, db)

            @pl.when(s + NBUF < NSTEP)
            def _():
                issue(s + NBUF, db)

        return carry

    lax.fori_loop(0, NSTEP // NBUF, step_quad, 0)
    pltpu.sync_copy(scores, out_h.at[pl.ds(base_b * 4, B_PER * 4), :])


def _sc_scores(tabE, tabR, sub, rel, ent4, dRI):
    mesh = plsc.VectorSubcoreMesh(core_axis_name="c", subcore_axis_name="s")
    f = pl.kernel(
        _sc_body,
        out_type=jax.ShapeDtypeStruct((BS * 4, 2 * D), jnp.float32),
        mesh=mesh,
        compiler_params=pltpu.CompilerParams(needs_layout_passes=False),
        scratch_types=[
            pltpu.VMEM((B_PER,), jnp.int32),
            pltpu.VMEM((B_PER,), jnp.int32),
            pltpu.VMEM((B_PER, 2 * D), jnp.float32),
            pltpu.VMEM((B_PER, 2 * D), jnp.float32),
            pltpu.VMEM((B_PER, 2 * D), jnp.float32),
            pltpu.VMEM((B_PER, 2 * D), jnp.float32),
            pltpu.VMEM((B_PER * 4, CHUNK), jnp.int32),
            pltpu.VMEM((NBUF, CHUNK, 2 * D), jnp.float32),
            pltpu.VMEM((B_PER * 4, 2 * D), jnp.float32),
            pltpu.SemaphoreType.DMA,
            pltpu.SemaphoreType.DMA,
            pltpu.SemaphoreType.DMA,
            pltpu.SemaphoreType.DMA,
            pltpu.SemaphoreType.DMA,
        ],
    )
    return f(tabE, tabR, sub, rel, ent4, dRI)


def _pack_body(rt_ref, it_ref, o_ref):
    o_ref[:, :D] = rt_ref[...].T
    o_ref[:, D:] = it_ref[...].T


def _pack_tables(real, img, blk):
    n = real.shape[0]
    grid = (n + blk - 1) // blk
    return pl.pallas_call(
        _pack_body,
        grid=(grid,),
        in_specs=[
            pl.BlockSpec((D, blk), lambda i: (0, i)),
            pl.BlockSpec((D, blk), lambda i: (0, i)),
        ],
        out_specs=pl.BlockSpec((blk, 2 * D), lambda i: (i, 0)),
        out_shape=jax.ShapeDtypeStruct((n, 2 * D), jnp.float32),
    )(real.T, img.T)


def _trig_body(day_ref, w1_ref, w2_ref, dri_ref):
    day = day_ref[...]
    dri_ref[:, :D] = jnp.cos(w2_ref[...] * day)
    dri_ref[:, D:] = jnp.sin(w1_ref[...] * day)


def _loss_body(x_ref, out_ref):
    x = x_ref[...]
    col = lax.broadcasted_iota(jnp.int32, (BS, NT), 1)
    valid = col < N_CAND
    xm = jnp.where(valid, x, -jnp.inf)
    m = jnp.max(xm, axis=1, keepdims=True)
    ssum = jnp.sum(jnp.where(valid, jnp.exp(xm - m), 0.0), axis=1,
                   keepdims=True)
    lse = m + jnp.log(ssum)
    pos = jnp.sum(jnp.where(col == 0, x, 0.0), axis=1, keepdims=True)
    val = jnp.sum(lse - pos) * (1.0 / BS)
    out_ref[...] = jnp.full((1, 1), val, jnp.float32)


def kernel(sub, rel, obj, year, month, day, neg, emb_E_real, emb_E_img,
           emb_R_real, emb_R_img, w1, w2):
    ent = jnp.concatenate([obj[:, None], neg], axis=1).astype(jnp.int32)
    ent = jnp.pad(ent, ((0, 0), (0, NT - N_CAND)))
    ent4 = ent.reshape(BS * 4, 2 * D)

    tabE = _pack_tables(emb_E_real, emb_E_img, 2048)
    tabR = _pack_tables(emb_R_real, emb_R_img, 1000)

    dRI = pl.pallas_call(
        _trig_body,
        out_shape=jax.ShapeDtypeStruct((BS, 2 * D), jnp.float32),
    )(day.reshape(BS, 1), w1.reshape(1, D), w2.reshape(1, D))

    scores4 = _sc_scores(tabE, tabR, sub.astype(jnp.int32),
                         rel.astype(jnp.int32), ent4, dRI)

    loss = pl.pallas_call(
        _loss_body,
        out_shape=jax.ShapeDtypeStruct((1, 1), jnp.float32),
    )(scores4.reshape(BS, NT))
    return loss[0, 0]


# speedup vs baseline: 1.1987x; 1.1967x over previous
import jax
import jax.numpy as jnp
from jax import lax
from jax.experimental import pallas as pl
from jax.experimental.pallas import tpu as pltpu
from jax.experimental.pallas import tpu_sc as plsc

BS = 1024
D = 64
N_CAND = 501
NT = 512
NW = 32
B_PER = BS // NW
CHUNK = 128
NBUF = 4
NQ = NT // CHUNK
NSTEP = B_PER * NQ
NLANE = 16
NC = D // NLANE


def _sc_body(tabE, tabR, sub_h, rel_h, ent_h, dRI_h, out_h,
             sub_v, rel_v, es, rr, dRI, aRI,
             idx_all, rows, scores, sem0, semg0, semg1, semg2, semg3):
    wid = lax.axis_index("s") * 2 + lax.axis_index("c")
    base_b = wid * B_PER

    pltpu.sync_copy(sub_h.at[pl.ds(base_b, B_PER)], sub_v)
    pltpu.sync_copy(rel_h.at[pl.ds(base_b, B_PER)], rel_v)
    pltpu.sync_copy(dRI_h.at[pl.ds(base_b, B_PER)], dRI)
    pltpu.sync_copy(ent_h.at[pl.ds(base_b * 4, B_PER * 4), :], idx_all)
    pltpu.async_copy(tabE.at[sub_v], es, sem0).wait()
    pltpu.async_copy(tabR.at[rel_v], rr, sem0).wait()

    def prep(i, carry):
        for c in range(NC):
            s = pl.ds(c * NLANE, NLANE)
            si = pl.ds(D + c * NLANE, NLANE)
            er = es[i, s]
            ei = es[i, si]
            dr = dRI[i, s]
            di = dRI[i, si]
            aRI[i, s] = er * dr - ei * di + rr[i, s]
            aRI[i, si] = er * di + ei * dr + rr[i, si]
        return carry

    lax.fori_loop(0, B_PER, prep, 0)

    lane = lax.iota(jnp.int32, NLANE)
    bufs = ((rows.at[0], semg0), (rows.at[1], semg1),
            (rows.at[2], semg2), (rows.at[3], semg3))

    def issue(step, db):
        rbuf, sem = bufs[db]
        pltpu.async_copy(tabE.at[idx_all.at[step]], rbuf, sem)

    for _p in range(NBUF):
        issue(_p, _p)

    def compute(step, db):
        b_local = lax.shift_right_logical(step, 2)
        q = lax.bitwise_and(step, NQ - 1)
        rbuf, sem = bufs[db]
        pltpu.make_async_copy(tabE.at[idx_all.at[step]], rbuf, sem).wait()

        ars = [aRI[b_local, pl.ds(c * NLANE, NLANE)] for c in range(NC)]
        ais = [aRI[b_local, pl.ds(D + c * NLANE, NLANE)] for c in range(NC)]
        drs = [dRI[b_local, pl.ds(c * NLANE, NLANE)] for c in range(NC)]
        dis = [dRI[b_local, pl.ds(D + c * NLANE, NLANE)] for c in range(NC)]

        def group(h, gcarry):
            vec = jnp.zeros((NLANE,), jnp.float32)
            for jj in range(NLANE):
                j = h * NLANE + jj
                part = []
                for c in range(NC):
                    er = rbuf[j, pl.ds(c * NLANE, NLANE)]
                    ei = rbuf[j, pl.ds(D + c * NLANE, NLANE)]
                    t1 = (ars[c] + ei * dis[c]) - er * drs[c]
                    t2 = (ais[c] + er * dis[c]) + ei * drs[c]
                    part.append(jnp.abs(t1) + jnp.abs(t2))
                acc = (part[0] + part[1]) + (part[2] + part[3])
                vec = jnp.where(lane == jj,
                                jnp.full((NLANE,), jnp.sum(acc)), vec)
            row = b_local * 4 + q
            scores[row, pl.ds(h * NLANE, NLANE)] = vec
            return gcarry

        lax.fori_loop(0, CHUNK // NLANE, group, 0)

    def step_quad(t, carry):
        for db in range(NBUF):
            s = NBUF * t + db
            compute(s, db)

            @pl.when(s + NBUF < NSTEP)
            def _():
                issue(s + NBUF, db)

        return carry

    lax.fori_loop(0, NSTEP // NBUF, step_quad, 0)
    pltpu.sync_copy(scores, out_h.at[pl.ds(base_b * 4, B_PER * 4), :])


def _sc_scores(tabE, tabR, sub, rel, ent4, dRI):
    mesh = plsc.VectorSubcoreMesh(core_axis_name="c", subcore_axis_name="s")
    f = pl.kernel(
        _sc_body,
        out_type=jax.ShapeDtypeStruct((BS * 4, 2 * D), jnp.float32),
        mesh=mesh,
        compiler_params=pltpu.CompilerParams(needs_layout_passes=False),
        scratch_types=[
            pltpu.VMEM((B_PER,), jnp.int32),
            pltpu.VMEM((B_PER,), jnp.int32),
            pltpu.VMEM((B_PER, 2 * D), jnp.float32),
            pltpu.VMEM((B_PER, 2 * D), jnp.float32),
            pltpu.VMEM((B_PER, 2 * D), jnp.float32),
            pltpu.VMEM((B_PER, 2 * D), jnp.float32),
            pltpu.VMEM((B_PER * 4, CHUNK), jnp.int32),
            pltpu.VMEM((NBUF, CHUNK, 2 * D), jnp.float32),
            pltpu.VMEM((B_PER * 4, 2 * D), jnp.float32),
            pltpu.SemaphoreType.DMA,
            pltpu.SemaphoreType.DMA,
            pltpu.SemaphoreType.DMA,
            pltpu.SemaphoreType.DMA,
            pltpu.SemaphoreType.DMA,
        ],
    )
    return f(tabE, tabR, sub, rel, ent4, dRI)


def _pack_body(rt_ref, it_ref, o_ref):
    o_ref[:, :D] = rt_ref[...].T
    o_ref[:, D:] = it_ref[...].T


def _pack_tables(real, img, blk):
    n = real.shape[0]
    grid = (n + blk - 1) // blk
    return pl.pallas_call(
        _pack_body,
        grid=(grid,),
        in_specs=[
            pl.BlockSpec((D, blk), lambda i: (0, i)),
            pl.BlockSpec((D, blk), lambda i: (0, i)),
        ],
        out_specs=pl.BlockSpec((blk, 2 * D), lambda i: (i, 0)),
        out_shape=jax.ShapeDtypeStruct((n, 2 * D), jnp.float32),
    )(real.T, img.T)


def _trig_body(day_ref, w1_ref, w2_ref, dri_ref):
    day = day_ref[...]
    dri_ref[:, :D] = jnp.cos(w2_ref[...] * day)
    dri_ref[:, D:] = jnp.sin(w1_ref[...] * day)


def _loss_body(x_ref, out_ref):
    x = x_ref[...]
    col = lax.broadcasted_iota(jnp.int32, (BS, NT), 1)
    valid = col < N_CAND
    xm = jnp.where(valid, x, -jnp.inf)
    m = jnp.max(xm, axis=1, keepdims=True)
    ssum = jnp.sum(jnp.where(valid, jnp.exp(xm - m), 0.0), axis=1,
                   keepdims=True)
    lse = m + jnp.log(ssum)
    pos = jnp.sum(jnp.where(col == 0, x, 0.0), axis=1, keepdims=True)
    val = jnp.sum(lse - pos) * (1.0 / BS)
    out_ref[...] = jnp.full((1, 1), val, jnp.float32)


def kernel(sub, rel, obj, year, month, day, neg, emb_E_real, emb_E_img,
           emb_R_real, emb_R_img, w1, w2):
    ent = jnp.concatenate([obj[:, None], neg], axis=1).astype(jnp.int32)
    ent = jnp.pad(ent, ((0, 0), (0, NT - N_CAND)))
    ent4 = ent.reshape(BS * 4, 2 * D)

    tabE = _pack_tables(emb_E_real, emb_E_img, 8192)
    tabR = _pack_tables(emb_R_real, emb_R_img, 1000)

    dRI = pl.pallas_call(
        _trig_body,
        out_shape=jax.ShapeDtypeStruct((BS, 2 * D), jnp.float32),
    )(day.reshape(BS, 1), w1.reshape(1, D), w2.reshape(1, D))

    scores4 = _sc_scores(tabE, tabR, sub.astype(jnp.int32),
                         rel.astype(jnp.int32), ent4, dRI)

    loss = pl.pallas_call(
        _loss_body,
        out_shape=jax.ShapeDtypeStruct((1, 1), jnp.float32),
    )(scores4.reshape(BS, NT))
    return loss[0, 0]


# speedup vs baseline: 1.2352x; 1.0304x over previous
import jax
import jax.numpy as jnp
from jax import lax
from jax.experimental import pallas as pl
from jax.experimental.pallas import tpu as pltpu
from jax.experimental.pallas import tpu_sc as plsc

BS = 1024
D = 64
N_CAND = 501
NT = 512
NW = 32
B_PER = BS // NW
CHUNK = 128
NBUF = 4
NQ = NT // CHUNK
NSTEP = B_PER * NQ
NLANE = 16
NC = D // NLANE


def _sc_body(tabE, tabR, sub_h, rel_h, ent_h, dRI_h, out_h,
             sub_v, rel_v, es, rr, dRI, aRI,
             idx_all, rows, scores, sem0, semg0, semg1, semg2, semg3):
    wid = lax.axis_index("s") * 2 + lax.axis_index("c")
    base_b = wid * B_PER

    pltpu.sync_copy(sub_h.at[pl.ds(base_b, B_PER)], sub_v)
    pltpu.sync_copy(rel_h.at[pl.ds(base_b, B_PER)], rel_v)
    pltpu.sync_copy(dRI_h.at[pl.ds(base_b, B_PER)], dRI)
    pltpu.sync_copy(ent_h.at[pl.ds(base_b * 4, B_PER * 4), :], idx_all)
    pltpu.async_copy(tabE.at[sub_v], es, sem0).wait()
    pltpu.async_copy(tabR.at[rel_v], rr, sem0).wait()

    def prep(i, carry):
        for c in range(NC):
            s = pl.ds(c * NLANE, NLANE)
            si = pl.ds(D + c * NLANE, NLANE)
            er = es[i, s]
            ei = es[i, si]
            dr = dRI[i, s]
            di = dRI[i, si]
            aRI[i, s] = er * dr - ei * di + rr[i, s]
            aRI[i, si] = er * di + ei * dr + rr[i, si]
        return carry

    lax.fori_loop(0, B_PER, prep, 0)

    lane = lax.iota(jnp.int32, NLANE)
    bufs = ((rows.at[0], semg0), (rows.at[1], semg1),
            (rows.at[2], semg2), (rows.at[3], semg3))

    def issue(step, db):
        rbuf, sem = bufs[db]
        pltpu.async_copy(tabE.at[idx_all.at[step]], rbuf, sem)

    for _p in range(NBUF):
        issue(_p, _p)

    def compute(step, db):
        b_local = lax.shift_right_logical(step, 2)
        q = lax.bitwise_and(step, NQ - 1)
        rbuf, sem = bufs[db]
        pltpu.make_async_copy(tabE.at[idx_all.at[step]], rbuf, sem).wait()

        ars = [aRI[b_local, pl.ds(c * NLANE, NLANE)] for c in range(NC)]
        ais = [aRI[b_local, pl.ds(D + c * NLANE, NLANE)] for c in range(NC)]
        drs = [dRI[b_local, pl.ds(c * NLANE, NLANE)] for c in range(NC)]
        dis = [dRI[b_local, pl.ds(D + c * NLANE, NLANE)] for c in range(NC)]

        def group(h, gcarry):
            vec = jnp.zeros((NLANE,), jnp.float32)
            for jj in range(NLANE):
                j = h * NLANE + jj
                part = []
                for c in range(NC):
                    er = rbuf[j, pl.ds(c * NLANE, NLANE)]
                    ei = rbuf[j, pl.ds(D + c * NLANE, NLANE)]
                    t1 = (ars[c] + ei * dis[c]) - er * drs[c]
                    t2 = (ais[c] + er * dis[c]) + ei * drs[c]
                    part.append(jnp.abs(t1) + jnp.abs(t2))
                acc = (part[0] + part[1]) + (part[2] + part[3])
                vec = jnp.where(lane == jj,
                                jnp.full((NLANE,), jnp.sum(acc)), vec)
            row = b_local * 4 + q
            scores[row, pl.ds(h * NLANE, NLANE)] = vec
            return gcarry

        lax.fori_loop(0, CHUNK // NLANE, group, 0)

    def step_quad(t, carry):
        for db in range(NBUF):
            s = NBUF * t + db
            compute(s, db)

            @pl.when(s + NBUF < NSTEP)
            def _():
                issue(s + NBUF, db)

        return carry

    lax.fori_loop(0, NSTEP // NBUF, step_quad, 0)
    pltpu.sync_copy(scores, out_h.at[pl.ds(base_b * 4, B_PER * 4), :])


def _sc_scores(tabE, tabR, sub, rel, ent4, dRI):
    mesh = plsc.VectorSubcoreMesh(core_axis_name="c", subcore_axis_name="s")
    f = pl.kernel(
        _sc_body,
        out_type=jax.ShapeDtypeStruct((BS * 4, 2 * D), jnp.float32),
        mesh=mesh,
        compiler_params=pltpu.CompilerParams(needs_layout_passes=False),
        scratch_types=[
            pltpu.VMEM((B_PER,), jnp.int32),
            pltpu.VMEM((B_PER,), jnp.int32),
            pltpu.VMEM((B_PER, 2 * D), jnp.float32),
            pltpu.VMEM((B_PER, 2 * D), jnp.float32),
            pltpu.VMEM((B_PER, 2 * D), jnp.float32),
            pltpu.VMEM((B_PER, 2 * D), jnp.float32),
            pltpu.VMEM((B_PER * 4, CHUNK), jnp.int32),
            pltpu.VMEM((NBUF, CHUNK, 2 * D), jnp.float32),
            pltpu.VMEM((B_PER * 4, 2 * D), jnp.float32),
            pltpu.SemaphoreType.DMA,
            pltpu.SemaphoreType.DMA,
            pltpu.SemaphoreType.DMA,
            pltpu.SemaphoreType.DMA,
            pltpu.SemaphoreType.DMA,
        ],
    )
    return f(tabE, tabR, sub, rel, ent4, dRI)


def _pack_body(rt_ref, it_ref, o_ref):
    o_ref[:, :D] = rt_ref[...].T
    o_ref[:, D:] = it_ref[...].T


def _pack_tables(real, img, blk):
    n = real.shape[0]
    grid = (n + blk - 1) // blk
    return pl.pallas_call(
        _pack_body,
        grid=(grid,),
        in_specs=[
            pl.BlockSpec((D, blk), lambda i: (0, i)),
            pl.BlockSpec((D, blk), lambda i: (0, i)),
        ],
        out_specs=pl.BlockSpec((blk, 2 * D), lambda i: (i, 0)),
        out_shape=jax.ShapeDtypeStruct((n, 2 * D), jnp.float32),
    )(real.T, img.T)


def _trig_body(day_ref, w1_ref, w2_ref, dri_ref):
    day = day_ref[...]
    dri_ref[:, :D] = jnp.cos(w2_ref[...] * day)
    dri_ref[:, D:] = jnp.sin(w1_ref[...] * day)


def _loss_body(x_ref, out_ref):
    x = x_ref[...]
    col = lax.broadcasted_iota(jnp.int32, (BS, NT), 1)
    valid = col < N_CAND
    xm = jnp.where(valid, x, -jnp.inf)
    m = jnp.max(xm, axis=1, keepdims=True)
    ssum = jnp.sum(jnp.where(valid, jnp.exp(xm - m), 0.0), axis=1,
                   keepdims=True)
    lse = m + jnp.log(ssum)
    pos = jnp.sum(jnp.where(col == 0, x, 0.0), axis=1, keepdims=True)
    val = jnp.sum(lse - pos) * (1.0 / BS)
    out_ref[...] = jnp.full((1, 1), val, jnp.float32)


def kernel(sub, rel, obj, year, month, day, neg, emb_E_real, emb_E_img,
           emb_R_real, emb_R_img, w1, w2):
    ent = jnp.concatenate([obj[:, None], neg], axis=1).astype(jnp.int32)
    ent = jnp.pad(ent, ((0, 0), (0, NT - N_CAND)))
    ent4 = ent.reshape(BS * 4, 2 * D)

    tabE = _pack_tables(emb_E_real, emb_E_img, 16384)
    tabR = _pack_tables(emb_R_real, emb_R_img, 1000)

    dRI = pl.pallas_call(
        _trig_body,
        out_shape=jax.ShapeDtypeStruct((BS, 2 * D), jnp.float32),
    )(day.reshape(BS, 1), w1.reshape(1, D), w2.reshape(1, D))

    scores4 = _sc_scores(tabE, tabR, sub.astype(jnp.int32),
                         rel.astype(jnp.int32), ent4, dRI)

    loss = pl.pallas_call(
        _loss_body,
        out_shape=jax.ShapeDtypeStruct((1, 1), jnp.float32),
    )(scores4.reshape(BS, NT))
    return loss[0, 0]
